# baseline (speedup 1.0000x reference)
import functools

import jax
import jax.numpy as jnp
from jax import lax
from jax.experimental import pallas as pl
from jax.experimental.pallas import tpu as pltpu
from jax.experimental.pallas import tpu_sc as plsc

NC = 2
NS = 16
NW = NC * NS
CH = 128


def _svd_sc(users, movies, U, M, bu, bm):
    B = users.shape[0]
    D = U.shape[1]
    bpw = B // NW
    nch = bpw // CH

    mesh = plsc.VectorSubcoreMesh(core_axis_name="c", subcore_axis_name="s")

    @functools.partial(
        pl.kernel,
        out_type=jax.ShapeDtypeStruct((B,), jnp.float32),
        mesh=mesh,
        scratch_types=[
            pltpu.VMEM((nch, CH), jnp.int32),
            pltpu.VMEM((nch, CH), jnp.int32),
            pltpu.VMEM((bpw, D), jnp.float32),
            pltpu.VMEM((bpw, D), jnp.float32),
            pltpu.VMEM((bpw,), jnp.float32),
            pltpu.VMEM((bpw,), jnp.float32),
            pltpu.VMEM((bpw,), jnp.float32),
            pltpu.SemaphoreType.DMA,
        ],
        compiler_params=pltpu.CompilerParams(use_tc_tiling_on_sc=False),
    )
    def body(users_hbm, movies_hbm, u_hbm, m_hbm, bu_hbm, bm_hbm, out_hbm,
             uidx, midx, urows, mrows, buv, bmv, outv, sem):
        wid = lax.axis_index("s") * NC + lax.axis_index("c")
        base = wid * bpw

        for j in range(nch):
            pltpu.sync_copy(users_hbm.at[pl.ds(base + j * CH, CH)], uidx.at[j])
            pltpu.sync_copy(movies_hbm.at[pl.ds(base + j * CH, CH)], midx.at[j])

        copies = []
        for j in range(nch):
            sl = pl.ds(j * CH, CH)
            copies.append(pltpu.async_copy(u_hbm.at[uidx.at[j]], urows.at[sl], sem))
            copies.append(pltpu.async_copy(m_hbm.at[midx.at[j]], mrows.at[sl], sem))
            copies.append(pltpu.async_copy(bu_hbm.at[uidx.at[j]], buv.at[sl], sem))
            copies.append(pltpu.async_copy(bm_hbm.at[midx.at[j]], bmv.at[sl], sem))
        for c in copies:
            c.wait()

        lane = lax.iota(jnp.int32, 16)
        perms = [lane ^ 8, lane ^ 4, lane ^ 2, lane ^ 1]

        def group(g, _):
            b0 = g * 16
            acc = buv[pl.ds(b0, 16)] + bmv[pl.ds(b0, 16)]
            for l in range(16):
                b = b0 + l
                v = (urows[b, pl.ds(0, 16)] * mrows[b, pl.ds(0, 16)]
                     + urows[b, pl.ds(16, 16)] * mrows[b, pl.ds(16, 16)])
                for p in perms:
                    v = v + v.at[p].get(mode="promise_in_bounds")
                acc = jnp.where(lane == l, acc + v, acc)
            outv[pl.ds(b0, 16)] = acc
            return ()

        lax.fori_loop(0, bpw // 16, group, ())

        pltpu.sync_copy(outv, out_hbm.at[pl.ds(base, bpw)])

    return body(users, movies, U, M, bu, bm)


def kernel(users, movies, U, M, bu, bm):
    r = _svd_sc(users, movies, U, M, bu, bm)
    return r.reshape(-1, 1)
